# baseline (speedup 1.0000x reference)
import functools

import jax
import jax.numpy as jnp
from jax import lax
from jax.experimental import pallas as pl
from jax.experimental.pallas import tpu as pltpu
from jax.experimental.pallas import tpu_sc as plsc

N_NODES = 10000
D = 128
DH = D // 2
N_LAYERS = 8
N_PAD = 10240
DUMMY = N_NODES

NC, NS = 2, 16
NW = NC * NS
CHUNK = 128
NCT = 168
NCHUNK_DEG = 84
E_PAD = NS * NCT * CHUNK
ROWS_PER_TILE = N_PAD // NS
NBUF = 4

_mesh = plsc.VectorSubcoreMesh(core_axis_name="c", subcore_axis_name="s")


@functools.partial(
    pl.kernel,
    mesh=_mesh,
    out_type=jax.ShapeDtypeStruct((NC, N_PAD), jnp.float32),
    scratch_types=[
        pltpu.VMEM((NCHUNK_DEG, CHUNK), jnp.int32),
        pltpu.VMEM((CHUNK,), jnp.float32),
        pltpu.VMEM((ROWS_PER_TILE,), jnp.float32),
        pltpu.VMEM_SHARED((N_PAD,), jnp.float32),
        pltpu.SemaphoreType.DMA,
    ],
)
def _deg_kernel(dst_hbm, out_hbm, didx_v, ones_v, node_v, acc, sem):
    c = lax.axis_index("c")
    s = lax.axis_index("s")
    for i in range(ROWS_PER_TILE // 16):
        node_v[pl.ds(16 * i, 16)] = jnp.zeros((16,), jnp.float32)
    for i in range(CHUNK // 16):
        ones_v[pl.ds(16 * i, 16)] = jnp.ones((16,), jnp.float32)

    nbase = s * ROWS_PER_TILE
    pltpu.sync_copy(node_v, acc.at[pl.ds(nbase, ROWS_PER_TILE)])
    w = c * NS + s
    pltpu.sync_copy(dst_hbm.at[w], didx_v)
    plsc.subcore_barrier()

    def chunk_body(j, carry):
        pltpu.async_copy(ones_v, acc.at[didx_v.at[j]], sem, add=True)

        @pl.when(j >= 4)
        def _():
            pltpu.make_async_copy(ones_v, acc.at[didx_v.at[j]], sem).wait()

        return carry

    lax.fori_loop(0, NCHUNK_DEG, chunk_body, 0)
    for j in range(4):
        pltpu.make_async_copy(ones_v, acc.at[didx_v.at[j]], sem).wait()
    plsc.subcore_barrier()
    pltpu.sync_copy(acc.at[pl.ds(nbase, ROWS_PER_TILE)], node_v)
    pltpu.sync_copy(node_v, out_hbm.at[c, pl.ds(nbase, ROWS_PER_TILE)])


@functools.partial(
    pl.kernel,
    mesh=_mesh,
    out_type=jax.ShapeDtypeStruct((NC, N_PAD, DH), jnp.float32),
    scratch_types=[
        pltpu.VMEM((NCT, CHUNK), jnp.int32),
        pltpu.VMEM((NCT, CHUNK), jnp.int32),
        pltpu.VMEM((NBUF, CHUNK, DH), jnp.float32),
        pltpu.VMEM_SHARED((N_PAD, DH), jnp.float32),
        pltpu.SemaphoreType.DMA((NBUF,)),
        pltpu.SemaphoreType.DMA((NBUF,)),
    ],
    compiler_params=pltpu.CompilerParams(use_tc_tiling_on_sc=False),
)
def _agg_kernel(y_hbm, src_hbm, dst_hbm, out_hbm, sidx_v, didx_v, rows_v, acc,
                gsem, ssem):
    c = lax.axis_index("c")
    s = lax.axis_index("s")
    tbl = y_hbm.at[c]

    def zero_row(i, carry):
        for k in range(DH // 16):
            rows_v[0, i, pl.ds(16 * k, 16)] = jnp.zeros((16,), jnp.float32)
        return carry

    lax.fori_loop(0, CHUNK, zero_row, 0)

    nbase = s * ROWS_PER_TILE
    for k in range(ROWS_PER_TILE // CHUNK):
        pltpu.sync_copy(rows_v.at[0], acc.at[pl.ds(nbase + k * CHUNK, CHUNK)])
    pltpu.sync_copy(src_hbm.at[s], sidx_v)
    pltpu.sync_copy(dst_hbm.at[s], didx_v)
    plsc.subcore_barrier()

    def start_gather(j, b):
        pltpu.async_copy(tbl.at[sidx_v.at[j]], rows_v.at[b], gsem.at[b])

    def wait_gather(j, b):
        pltpu.make_async_copy(
            tbl.at[sidx_v.at[j]], rows_v.at[b], gsem.at[b]).wait()

    def start_scatter(j, b):
        pltpu.async_copy(rows_v.at[b], acc.at[didx_v.at[j]], ssem.at[b],
                         add=True)

    def wait_scatter(j, b):
        pltpu.make_async_copy(
            rows_v.at[b], acc.at[didx_v.at[j]], ssem.at[b]).wait()

    start_gather(0, 0)
    start_gather(1, 1)
    start_gather(2, 2)
    wait_gather(0, 0)
    start_scatter(0, 0)
    start_gather(3, 3)
    wait_gather(1, 1)
    start_scatter(1, 1)

    def pipe_body(i, carry):
        for b in range(NBUF):
            j = NBUF * i + NBUF + b
            jm2 = j - 2
            b2 = (b + 2) % NBUF
            wait_scatter(j - NBUF, b)
            start_gather(j, b)
            wait_gather(jm2, b2)
            start_scatter(jm2, b2)
        return carry

    lax.fori_loop(0, (NCT - NBUF) // NBUF, pipe_body, 0)

    wait_gather(NCT - 2, (NCT - 2) % NBUF)
    start_scatter(NCT - 2, (NCT - 2) % NBUF)
    wait_gather(NCT - 1, (NCT - 1) % NBUF)
    start_scatter(NCT - 1, (NCT - 1) % NBUF)
    for b in range(NBUF):
        wait_scatter(NCT - NBUF + b, (NCT - NBUF + b) % NBUF)
    plsc.subcore_barrier()
    for k in range(ROWS_PER_TILE // CHUNK):
        pltpu.sync_copy(acc.at[pl.ds(nbase + k * CHUNK, CHUNK)], rows_v.at[0])
        pltpu.sync_copy(rows_v.at[0],
                        out_hbm.at[c, pl.ds(nbase + k * CHUNK, CHUNK)])


def _dis(deg):
    return jnp.where(deg > 0, lax.rsqrt(jnp.maximum(deg, 1e-12)), 0.0)


def _scale_body(x_ref, d0_ref, d1_ref, y_ref):
    deg = d0_ref[...] + d1_ref[...]
    y = x_ref[...] * _dis(deg)
    y_ref[...] = y.reshape(_ROW_BLK, NC, DH).swapaxes(0, 1)


_ROW_BLK = 512
_N_BLKS = N_PAD // _ROW_BLK


def _scale_call(xp, d0, d1):
    return pl.pallas_call(
        _scale_body,
        grid=(_N_BLKS,),
        in_specs=[
            pl.BlockSpec((_ROW_BLK, D), lambda i: (i, 0)),
            pl.BlockSpec((_ROW_BLK, 1), lambda i: (i, 0)),
            pl.BlockSpec((_ROW_BLK, 1), lambda i: (i, 0)),
        ],
        out_specs=pl.BlockSpec((NC, _ROW_BLK, DH), lambda i: (0, i, 0)),
        out_shape=jax.ShapeDtypeStruct((NC, N_PAD, DH), jnp.float32),
    )(xp, d0, d1)


def _mm_body(z0_ref, z1_ref, d0_ref, d1_ref, w_ref, b_ref, o_ref):
    deg = d0_ref[...] + d1_ref[...]
    dis = _dis(deg)
    xa = jnp.concatenate([z0_ref[...], z1_ref[...]], axis=1) * dis
    acc = lax.dot_general(
        xa, w_ref[...], (((1,), (0,)), ((), ())),
        preferred_element_type=jnp.float32,
    )
    o_ref[...] = jax.nn.sigmoid(acc + b_ref[...])


def _mm_call(z0, z1, d0, d1, w_cat, b_cat):
    return pl.pallas_call(
        _mm_body,
        grid=(_N_BLKS,),
        in_specs=[
            pl.BlockSpec((_ROW_BLK, DH), lambda i: (i, 0)),
            pl.BlockSpec((_ROW_BLK, DH), lambda i: (i, 0)),
            pl.BlockSpec((_ROW_BLK, 1), lambda i: (i, 0)),
            pl.BlockSpec((_ROW_BLK, 1), lambda i: (i, 0)),
            pl.BlockSpec((D, N_LAYERS * D), lambda i: (0, 0)),
            pl.BlockSpec((1, N_LAYERS * D), lambda i: (0, 0)),
        ],
        out_specs=pl.BlockSpec((_ROW_BLK, N_LAYERS * D), lambda i: (i, 0)),
        out_shape=jax.ShapeDtypeStruct((N_PAD, N_LAYERS * D), jnp.float32),
    )(z0, z1, d0, d1, w_cat, b_cat)


@jax.jit
def kernel(x, edge_index, W, b):
    loop = jnp.arange(N_NODES, dtype=edge_index.dtype)
    n_extra = E_PAD - edge_index.shape[1] - N_NODES
    src = jnp.concatenate(
        [edge_index[0], loop, jnp.zeros((n_extra,), edge_index.dtype)])
    dst = jnp.concatenate(
        [edge_index[1], loop, jnp.full((n_extra,), DUMMY, edge_index.dtype)])
    src_agg = src.reshape(NS, NCT, CHUNK)
    dst_agg = dst.reshape(NS, NCT, CHUNK)
    dst_deg = dst.reshape(NW, NCHUNK_DEG, CHUNK)

    degp = _deg_kernel(dst_deg)
    d0 = degp[0].reshape(N_PAD, 1)
    d1 = degp[1].reshape(N_PAD, 1)

    xp = jnp.pad(x, ((0, N_PAD - N_NODES), (0, 0)))
    y = _scale_call(xp, d0, d1)

    z = _agg_kernel(y, src_agg, dst_agg)

    w_cat = jnp.transpose(W, (1, 0, 2)).reshape(D, N_LAYERS * D)
    b_cat = b.reshape(1, N_LAYERS * D)
    out = _mm_call(z[0], z[1], d0, d1, w_cat, b_cat)
    return out[:N_NODES]
